# Initial kernel scaffold; baseline (speedup 1.0000x reference)
import jax
import jax.numpy as jnp
from jax.experimental import pallas as pl


def kernel(x, edge_index, Wl1, bl1, Wr1, Wl2, bl2, Wr2, Wl3, bl3, Wr3):
    raise NotImplementedError("write your pallas kernel here")


# speedup vs baseline: 2.4349x; 2.4349x over previous
import functools

import jax
import jax.numpy as jnp
from jax import lax
from jax.experimental import pallas as pl
from jax.experimental.pallas import tpu as pltpu
from jax.experimental.pallas import tpu_sc as plsc

L = 16
IB = 128
SPB = 8
NTILES = 16
WB = 1000


def _sc_agg_build(n, n_acc, eb, c_total, chunks_per_core, with_deg):
    blocks = eb // SPB
    per_tile = blocks // NTILES
    deg_per_tile = blocks // (2 * NTILES)
    rows_per_tile = n // NTILES
    n_sub = rows_per_tile // WB

    mesh = plsc.VectorSubcoreMesh(core_axis_name="c", subcore_axis_name="s")

    out_type = [jax.ShapeDtypeStruct((c_total * n, L), jnp.float32)]
    if with_deg:
        out_type.append(jax.ShapeDtypeStruct((2 * n, L), jnp.float32))

    scratch = [
        pltpu.VMEM_SHARED((n_acc, L), jnp.float32),
        pltpu.VMEM((SPB, IB, L), jnp.float32),
        pltpu.VMEM((SPB, IB), jnp.int32),
        pltpu.VMEM((SPB, IB), jnp.int32),
        pltpu.VMEM((IB, L), jnp.float32),
        pltpu.SemaphoreType.DMA,
    ]

    def body(hflat, gidx, dstb, zeros_hbm, ones_hbm, *refs):
        if with_deg:
            agg_out, deg_out = refs[0], refs[1]
            scr = refs[2:]
        else:
            agg_out = refs[0]
            deg_out = None
            scr = refs[1:]
        acc, rows_v, gi_v, di_v, ones_v, gsem = scr

        ci = lax.axis_index("c")
        sid = lax.axis_index("s")
        my_rows = sid * rows_per_tile

        if with_deg:
            pltpu.sync_copy(ones_hbm, ones_v)

        def zero_acc():
            for k in range(n_sub):
                pltpu.sync_copy(zeros_hbm, acc.at[pl.ds(my_rows + k * WB, WB)])

        def writeout(out_ref, out_base):
            for k in range(n_sub):
                pltpu.sync_copy(acc.at[pl.ds(my_rows + k * WB, WB)],
                                out_ref.at[pl.ds(out_base + my_rows + k * WB, WB)])

        for p in range(chunks_per_core):
            chunk = ci * chunks_per_core + p
            zero_acc()
            plsc.subcore_barrier()

            def blk_body(b, _, chunk=chunk):
                r0 = (sid * per_tile + b) * SPB
                pltpu.sync_copy(gidx.at[pl.ds(chunk * eb + r0, SPB)], gi_v)
                pltpu.sync_copy(dstb.at[pl.ds(r0, SPB)], di_v)
                descs = [
                    pltpu.async_copy(hflat.at[gi_v.at[j]], rows_v.at[j], gsem)
                    for j in range(SPB)
                ]
                for d in descs:
                    d.wait()
                for j in range(SPB):
                    pltpu.sync_copy(rows_v.at[j], acc.at[di_v.at[j]], add=True)
                return 0

            lax.fori_loop(0, per_tile, blk_body, 0)
            plsc.subcore_barrier()
            writeout(agg_out, chunk * n)
            plsc.subcore_barrier()

        if with_deg:
            zero_acc()
            plsc.subcore_barrier()

            def deg_body(b, _):
                r0 = (ci * blocks // 2 + sid * deg_per_tile + b) * SPB
                pltpu.sync_copy(dstb.at[pl.ds(r0, SPB)], di_v)
                for j in range(SPB):
                    pltpu.sync_copy(ones_v, acc.at[di_v.at[j]], add=True)
                return 0

            lax.fori_loop(0, deg_per_tile, deg_body, 0)
            plsc.subcore_barrier()
            writeout(deg_out, ci * n)

    return pl.kernel(
        body, out_type=out_type, mesh=mesh, scratch_types=scratch,
        compiler_params=pltpu.CompilerParams(use_tc_tiling_on_sc=False))


def _idx_prep(srcb, eb):
    RB = 256
    grid = eb // RB

    def body(src_ref, i8_ref, i2_ref):
        s = src_ref[...]
        c8 = lax.broadcasted_iota(jnp.int32, (8, RB, IB), 0)
        i8_ref[...] = s[None] * 8 + c8
        c2 = lax.broadcasted_iota(jnp.int32, (2, RB, IB), 0)
        i2_ref[...] = s[None] * 2 + c2

    i8, i2 = pl.pallas_call(
        body,
        grid=(grid,),
        in_specs=[pl.BlockSpec((RB, IB), lambda i: (i, 0))],
        out_specs=[
            pl.BlockSpec((8, RB, IB), lambda i: (0, i, 0)),
            pl.BlockSpec((2, RB, IB), lambda i: (0, i, 0)),
        ],
        out_shape=[
            jax.ShapeDtypeStruct((8, eb, IB), jnp.int32),
            jax.ShapeDtypeStruct((2, eb, IB), jnp.int32),
        ],
    )(srcb)
    return i8.reshape(8 * eb, IB), i2.reshape(2 * eb, IB)


def _mlp(agg, h, deg, wl, wr, bl, n):
    c_in = agg.shape[0]
    d_in = h.shape[1]
    BN = 2000
    grid = n // BN

    def body(agg_ref, h_ref, deg_ref, wl_ref, wr_ref, bl_ref, o_ref):
        deg_sum = deg_ref[0, :, 0:1] + deg_ref[1, :, 0:1]
        recip = 1.0 / jnp.maximum(deg_sum, 1.0)
        o = jnp.dot(h_ref[...], wr_ref[...], preferred_element_type=jnp.float32)
        for c in range(c_in):
            a = agg_ref[c] * recip
            o += jnp.dot(a, wl_ref[c * L:(c + 1) * L, :],
                         preferred_element_type=jnp.float32)
        o_ref[...] = jnp.maximum(o + bl_ref[...], 0.0)

    return pl.pallas_call(
        body,
        grid=(grid,),
        in_specs=[
            pl.BlockSpec((c_in, BN, L), lambda i: (0, i, 0)),
            pl.BlockSpec((BN, d_in), lambda i: (i, 0)),
            pl.BlockSpec((2, BN, L), lambda i: (0, i, 0)),
            pl.BlockSpec((c_in * L, 128), lambda i: (0, 0)),
            pl.BlockSpec((d_in, 128), lambda i: (0, 0)),
            pl.BlockSpec((1, 128), lambda i: (0, 0)),
        ],
        out_specs=pl.BlockSpec((BN, 128), lambda i: (i, 0)),
        out_shape=jax.ShapeDtypeStruct((n, 128), jnp.float32),
    )(agg, h, deg, wl, wr, bl)


def kernel(x, edge_index, Wl1, bl1, Wr1, Wl2, bl2, Wr2, Wl3, bl3, Wr3):
    n = x.shape[0]
    e = edge_index.shape[1]
    hid = Wl1.shape[1]

    unit = 2 * NTILES * SPB * IB
    e_pad = -(-e // unit) * unit
    eb = e_pad // IB
    pad = e_pad - e
    src_p = jnp.concatenate([edge_index[0], jnp.zeros((pad,), jnp.int32)])
    dst_p = jnp.concatenate([edge_index[1], jnp.full((pad,), n, jnp.int32)])
    srcb = src_p.reshape(eb, IB)
    dstb = dst_p.reshape(eb, IB)
    n_acc = n + L

    idx8, idx2 = _idx_prep(srcb, eb)
    zeros_hbm = jnp.zeros((WB, L), jnp.float32)
    ones_hbm = jnp.ones((IB, L), jnp.float32)

    d_pad = 2 * L
    x_pad = jnp.pad(x, ((0, 0), (0, d_pad - x.shape[1])))
    wl1p = jnp.pad(Wl1, ((0, d_pad - Wl1.shape[0]), (0, 0)))
    wr1p = jnp.pad(Wr1, ((0, d_pad - Wr1.shape[0]), (0, 0)))

    sc_l1 = _sc_agg_build(n, n_acc, eb, 2, 1, True)
    sc_conv = _sc_agg_build(n, n_acc, eb, 8, 4, False)

    agg1f, degf = sc_l1(x_pad.reshape(n * 2, L), idx2, dstb, zeros_hbm, ones_hbm)
    agg1 = agg1f.reshape(2, n, L)
    deg = degf.reshape(2, n, L)
    h1 = _mlp(agg1, x_pad, deg, wl1p, wr1p, bl1.reshape(1, hid), n)

    agg2 = sc_conv(h1.reshape(n * 8, L), idx8, dstb, zeros_hbm, ones_hbm)[0]
    h2 = _mlp(agg2.reshape(8, n, L), h1, deg, Wl2, Wr2, bl2.reshape(1, hid), n)

    agg3 = sc_conv(h2.reshape(n * 8, L), idx8, dstb, zeros_hbm, ones_hbm)[0]
    h3 = _mlp(agg3.reshape(8, n, L), h2, deg, Wl3, Wr3, bl3.reshape(1, hid), n)
    return h3


# speedup vs baseline: 2.7431x; 1.1265x over previous
import functools

import jax
import jax.numpy as jnp
from jax import lax
from jax.experimental import pallas as pl
from jax.experimental.pallas import tpu as pltpu
from jax.experimental.pallas import tpu_sc as plsc

L = 16
IB = 128
SPB = 8
NTILES = 16
WB = 1000
ZB = 250


def _sc_agg_build(n, n_acc, eb, c_total, chunks_per_core, with_deg):
    blocks = eb // SPB
    per_tile = blocks // NTILES
    deg_per_tile = blocks // (2 * NTILES)
    rows_per_tile = n // NTILES
    n_sub = rows_per_tile // WB
    n_sub_z = rows_per_tile // ZB

    mesh = plsc.VectorSubcoreMesh(core_axis_name="c", subcore_axis_name="s")

    out_type = [jax.ShapeDtypeStruct((c_total * n, L), jnp.float32)]
    if with_deg:
        out_type.append(jax.ShapeDtypeStruct((2 * n, L), jnp.float32))

    scratch = [
        pltpu.VMEM_SHARED((n_acc, L), jnp.float32),
        pltpu.VMEM((SPB, IB, L), jnp.float32),
        pltpu.VMEM((SPB, IB, L), jnp.float32),
        pltpu.VMEM((SPB, IB), jnp.int32),
        pltpu.VMEM((SPB, IB), jnp.int32),
        pltpu.VMEM((SPB, IB), jnp.int32),
        pltpu.VMEM((SPB, IB), jnp.int32),
        pltpu.VMEM((IB, L), jnp.float32),
        pltpu.VMEM((ZB, L), jnp.float32),
        pltpu.SemaphoreType.DMA,
        pltpu.SemaphoreType.DMA,
    ]

    def body(hflat, gidx, dstb, zeros_hbm, ones_hbm, *refs):
        if with_deg:
            agg_out, deg_out = refs[0], refs[1]
            scr = refs[2:]
        else:
            agg_out = refs[0]
            deg_out = None
            scr = refs[1:]
        acc, r0v, r1v, g0v, g1v, d0v, d1v, ones_v, zeros_v, sem0, sem1 = scr
        bufs = ((r0v, g0v, d0v, sem0), (r1v, g1v, d1v, sem1))

        ci = lax.axis_index("c")
        sid = lax.axis_index("s")
        my_rows = sid * rows_per_tile

        pltpu.sync_copy(zeros_hbm, zeros_v)
        if with_deg:
            pltpu.sync_copy(ones_hbm, ones_v)

        def zero_acc():
            for k in range(n_sub_z):
                pltpu.sync_copy(zeros_v, acc.at[pl.ds(my_rows + k * ZB, ZB)])

        def writeout(out_ref, out_base):
            for k in range(n_sub):
                pltpu.sync_copy(acc.at[pl.ds(my_rows + k * WB, WB)],
                                out_ref.at[pl.ds(out_base + my_rows + k * WB, WB)])

        def fire(r0, buf, chunk):
            rows, gi, di, sem = buf
            pltpu.sync_copy(gidx.at[pl.ds(chunk * eb + r0, SPB)], gi)
            pltpu.sync_copy(dstb.at[pl.ds(r0, SPB)], di)
            for j in range(SPB):
                pltpu.async_copy(hflat.at[gi.at[j]], rows.at[j], sem)

        def drain_scatter(buf):
            rows, gi, di, sem = buf
            for j in range(SPB):
                pltpu.make_async_copy(hflat.at[gi.at[j]], rows.at[j], sem).wait()
            for j in range(SPB):
                pltpu.sync_copy(rows.at[j], acc.at[di.at[j]], add=True)

        for p in range(chunks_per_core):
            chunk = ci * chunks_per_core + p
            zero_acc()
            plsc.subcore_barrier()
            base = sid * per_tile * SPB

            fire(base, bufs[0], chunk)

            def pair_body(g, _, chunk=chunk, base=base):
                b0r = base + 2 * g * SPB
                fire(b0r + SPB, bufs[1], chunk)
                drain_scatter(bufs[0])

                @pl.when(2 * g + 2 < per_tile)
                def _():
                    fire(b0r + 2 * SPB, bufs[0], chunk)

                drain_scatter(bufs[1])
                return 0

            lax.fori_loop(0, per_tile // 2, pair_body, 0)
            plsc.subcore_barrier()
            writeout(agg_out, chunk * n)
            plsc.subcore_barrier()

        if with_deg:
            zero_acc()
            plsc.subcore_barrier()

            def deg_body(b, _):
                r0 = (ci * blocks // 2 + sid * deg_per_tile + b) * SPB
                pltpu.sync_copy(dstb.at[pl.ds(r0, SPB)], d0v)
                for j in range(SPB):
                    pltpu.sync_copy(ones_v, acc.at[d0v.at[j]], add=True)
                return 0

            lax.fori_loop(0, deg_per_tile, deg_body, 0)
            plsc.subcore_barrier()
            writeout(deg_out, ci * n)

    return pl.kernel(
        body, out_type=out_type, mesh=mesh, scratch_types=scratch,
        compiler_params=pltpu.CompilerParams(use_tc_tiling_on_sc=False))


def _idx_prep(srcb, eb):
    RB = 256
    grid = eb // RB

    def body(src_ref, i8_ref, i2_ref):
        s = src_ref[...]
        c8 = lax.broadcasted_iota(jnp.int32, (8, RB, IB), 0)
        i8_ref[...] = s[None] * 8 + c8
        c2 = lax.broadcasted_iota(jnp.int32, (2, RB, IB), 0)
        i2_ref[...] = s[None] * 2 + c2

    i8, i2 = pl.pallas_call(
        body,
        grid=(grid,),
        in_specs=[pl.BlockSpec((RB, IB), lambda i: (i, 0))],
        out_specs=[
            pl.BlockSpec((8, RB, IB), lambda i: (0, i, 0)),
            pl.BlockSpec((2, RB, IB), lambda i: (0, i, 0)),
        ],
        out_shape=[
            jax.ShapeDtypeStruct((8, eb, IB), jnp.int32),
            jax.ShapeDtypeStruct((2, eb, IB), jnp.int32),
        ],
    )(srcb)
    return i8.reshape(8 * eb, IB), i2.reshape(2 * eb, IB)


def _mlp(agg, h, deg, wl, wr, bl, n):
    c_in = agg.shape[0]
    d_in = h.shape[1]
    BN = 2000
    grid = n // BN

    def body(agg_ref, h_ref, deg_ref, wl_ref, wr_ref, bl_ref, o_ref):
        deg_sum = deg_ref[0, :, 0:1] + deg_ref[1, :, 0:1]
        recip = 1.0 / jnp.maximum(deg_sum, 1.0)
        o = jnp.dot(h_ref[...], wr_ref[...], preferred_element_type=jnp.float32)
        for c in range(c_in):
            a = agg_ref[c] * recip
            o += jnp.dot(a, wl_ref[c * L:(c + 1) * L, :],
                         preferred_element_type=jnp.float32)
        o_ref[...] = jnp.maximum(o + bl_ref[...], 0.0)

    return pl.pallas_call(
        body,
        grid=(grid,),
        in_specs=[
            pl.BlockSpec((c_in, BN, L), lambda i: (0, i, 0)),
            pl.BlockSpec((BN, d_in), lambda i: (i, 0)),
            pl.BlockSpec((2, BN, L), lambda i: (0, i, 0)),
            pl.BlockSpec((c_in * L, 128), lambda i: (0, 0)),
            pl.BlockSpec((d_in, 128), lambda i: (0, 0)),
            pl.BlockSpec((1, 128), lambda i: (0, 0)),
        ],
        out_specs=pl.BlockSpec((BN, 128), lambda i: (i, 0)),
        out_shape=jax.ShapeDtypeStruct((n, 128), jnp.float32),
    )(agg, h, deg, wl, wr, bl)


def kernel(x, edge_index, Wl1, bl1, Wr1, Wl2, bl2, Wr2, Wl3, bl3, Wr3):
    n = x.shape[0]
    e = edge_index.shape[1]
    hid = Wl1.shape[1]

    unit = 2 * NTILES * SPB * IB
    e_pad = -(-e // unit) * unit
    eb = e_pad // IB
    pad = e_pad - e
    src_p = jnp.concatenate([edge_index[0], jnp.zeros((pad,), jnp.int32)])
    dst_p = jnp.concatenate([edge_index[1], jnp.full((pad,), n, jnp.int32)])
    srcb = src_p.reshape(eb, IB)
    dstb = dst_p.reshape(eb, IB)
    n_acc = n + L

    idx8, idx2 = _idx_prep(srcb, eb)
    zeros_hbm = jnp.zeros((ZB, L), jnp.float32)
    ones_hbm = jnp.ones((IB, L), jnp.float32)

    d_pad = 2 * L
    x_pad = jnp.pad(x, ((0, 0), (0, d_pad - x.shape[1])))
    wl1p = jnp.pad(Wl1, ((0, d_pad - Wl1.shape[0]), (0, 0)))
    wr1p = jnp.pad(Wr1, ((0, d_pad - Wr1.shape[0]), (0, 0)))

    sc_l1 = _sc_agg_build(n, n_acc, eb, 2, 1, True)
    sc_conv = _sc_agg_build(n, n_acc, eb, 8, 4, False)

    agg1f, degf = sc_l1(x_pad.reshape(n * 2, L), idx2, dstb, zeros_hbm, ones_hbm)
    agg1 = agg1f.reshape(2, n, L)
    deg = degf.reshape(2, n, L)
    h1 = _mlp(agg1, x_pad, deg, wl1p, wr1p, bl1.reshape(1, hid), n)

    agg2 = sc_conv(h1.reshape(n * 8, L), idx8, dstb, zeros_hbm, ones_hbm)[0]
    h2 = _mlp(agg2.reshape(8, n, L), h1, deg, Wl2, Wr2, bl2.reshape(1, hid), n)

    agg3 = sc_conv(h2.reshape(n * 8, L), idx8, dstb, zeros_hbm, ones_hbm)[0]
    h3 = _mlp(agg3.reshape(8, n, L), h2, deg, Wl3, Wr3, bl3.reshape(1, hid), n)
    return h3


# speedup vs baseline: 4.1552x; 1.5148x over previous
import functools

import jax
import jax.numpy as jnp
from jax import lax
from jax.experimental import pallas as pl
from jax.experimental.pallas import tpu as pltpu
from jax.experimental.pallas import tpu_sc as plsc

L = 16
IB = 128
SPB = 8
NTILES = 16
WB = 1000
ZB = 250


def _sc_agg_build(n, n_acc, eb, c_total, chunks_per_core, with_deg,
                  lanes=L, dtype=jnp.float32):
    blocks = eb // SPB
    per_tile = blocks // NTILES
    deg_per_tile = blocks // (2 * NTILES)
    rows_per_tile = n // NTILES
    n_sub = rows_per_tile // WB
    n_sub_z = rows_per_tile // ZB

    mesh = plsc.VectorSubcoreMesh(core_axis_name="c", subcore_axis_name="s")

    out_type = [jax.ShapeDtypeStruct((c_total * n, lanes), dtype)]
    if with_deg:
        out_type.append(jax.ShapeDtypeStruct((2 * n, L), jnp.float32))

    scratch = [
        pltpu.VMEM_SHARED((n_acc, lanes), dtype),
        pltpu.VMEM((SPB, IB, lanes), dtype),
        pltpu.VMEM((SPB, IB, lanes), dtype),
        pltpu.VMEM((SPB, IB), jnp.int32),
        pltpu.VMEM((SPB, IB), jnp.int32),
        pltpu.VMEM((SPB, IB), jnp.int32),
        pltpu.VMEM((SPB, IB), jnp.int32),
        pltpu.VMEM((IB, L), jnp.float32),
        pltpu.VMEM((ZB, lanes), dtype),
        pltpu.SemaphoreType.DMA,
        pltpu.SemaphoreType.DMA,
    ]

    def body(hflat, gidx, dstb, zeros_hbm, ones_hbm, *refs):
        if with_deg:
            agg_out, deg_out = refs[0], refs[1]
            scr = refs[2:]
        else:
            agg_out = refs[0]
            deg_out = None
            scr = refs[1:]
        acc, r0v, r1v, g0v, g1v, d0v, d1v, ones_v, zeros_v, sem0, sem1 = scr
        bufs = ((r0v, g0v, d0v, sem0), (r1v, g1v, d1v, sem1))

        ci = lax.axis_index("c")
        sid = lax.axis_index("s")
        my_rows = sid * rows_per_tile

        pltpu.sync_copy(zeros_hbm, zeros_v)
        if with_deg:
            pltpu.sync_copy(ones_hbm, ones_v)

        def zero_acc():
            for k in range(n_sub_z):
                pltpu.sync_copy(zeros_v, acc.at[pl.ds(my_rows + k * ZB, ZB)])

        def writeout(out_ref, out_base):
            for k in range(n_sub):
                pltpu.sync_copy(acc.at[pl.ds(my_rows + k * WB, WB)],
                                out_ref.at[pl.ds(out_base + my_rows + k * WB, WB)])

        def fire(r0, buf, chunk):
            rows, gi, di, sem = buf
            pltpu.sync_copy(gidx.at[pl.ds(chunk * eb + r0, SPB)], gi)
            pltpu.sync_copy(dstb.at[pl.ds(r0, SPB)], di)
            for j in range(SPB):
                pltpu.async_copy(hflat.at[gi.at[j]], rows.at[j], sem)

        def drain_scatter(buf):
            rows, gi, di, sem = buf
            for j in range(SPB):
                pltpu.make_async_copy(hflat.at[gi.at[j]], rows.at[j], sem).wait()
            for j in range(SPB):
                pltpu.sync_copy(rows.at[j], acc.at[di.at[j]], add=True)

        for p in range(chunks_per_core):
            chunk = ci * chunks_per_core + p
            zero_acc()
            plsc.subcore_barrier()
            base = sid * per_tile * SPB

            fire(base, bufs[0], chunk)

            def pair_body(g, _, chunk=chunk, base=base):
                b0r = base + 2 * g * SPB
                fire(b0r + SPB, bufs[1], chunk)
                drain_scatter(bufs[0])

                @pl.when(2 * g + 2 < per_tile)
                def _():
                    fire(b0r + 2 * SPB, bufs[0], chunk)

                drain_scatter(bufs[1])
                return 0

            lax.fori_loop(0, per_tile // 2, pair_body, 0)
            plsc.subcore_barrier()
            writeout(agg_out, chunk * n)
            plsc.subcore_barrier()

        if with_deg:
            zero_acc()
            plsc.subcore_barrier()

            def deg_body(b, _):
                r0 = (ci * blocks // 2 + sid * deg_per_tile + b) * SPB
                pltpu.sync_copy(dstb.at[pl.ds(r0, SPB)], d0v)
                for j in range(SPB):
                    pltpu.sync_copy(ones_v, acc.at[d0v.at[j]], add=True)
                return 0

            lax.fori_loop(0, deg_per_tile, deg_body, 0)
            plsc.subcore_barrier()
            writeout(deg_out, ci * n)

    return pl.kernel(
        body, out_type=out_type, mesh=mesh, scratch_types=scratch,
        compiler_params=pltpu.CompilerParams(use_tc_tiling_on_sc=False))


def _idx_prep(srcb, eb):
    RB = 256
    grid = eb // RB

    def body(src_ref, i4_ref, i2_ref):
        s = src_ref[...]
        c4 = lax.broadcasted_iota(jnp.int32, (4, RB, IB), 0)
        i4_ref[...] = s[None] * 4 + c4
        c2 = lax.broadcasted_iota(jnp.int32, (2, RB, IB), 0)
        i2_ref[...] = s[None] * 2 + c2

    i4, i2 = pl.pallas_call(
        body,
        grid=(grid,),
        in_specs=[pl.BlockSpec((RB, IB), lambda i: (i, 0))],
        out_specs=[
            pl.BlockSpec((4, RB, IB), lambda i: (0, i, 0)),
            pl.BlockSpec((2, RB, IB), lambda i: (0, i, 0)),
        ],
        out_shape=[
            jax.ShapeDtypeStruct((4, eb, IB), jnp.int32),
            jax.ShapeDtypeStruct((2, eb, IB), jnp.int32),
        ],
    )(srcb)
    return i4.reshape(4 * eb, IB), i2.reshape(2 * eb, IB)


def _mlp(agg, h, deg, wl, wr, bl, n, emit_bf16):
    c_in, _, lanes = agg.shape
    d_in = h.shape[1]
    BN = 2000
    grid = n // BN

    def body(agg_ref, h_ref, deg_ref, wl_ref, wr_ref, bl_ref, *outs):
        deg_sum = deg_ref[0, :, 0:1] + deg_ref[1, :, 0:1]
        recip = 1.0 / jnp.maximum(deg_sum, 1.0)
        o = jnp.dot(h_ref[...], wr_ref[...], preferred_element_type=jnp.float32)
        for c in range(c_in):
            a = agg_ref[c].astype(jnp.float32) * recip
            o += jnp.dot(a, wl_ref[c * lanes:(c + 1) * lanes, :],
                         preferred_element_type=jnp.float32)
        o = jnp.maximum(o + bl_ref[...], 0.0)
        outs[0][...] = o
        if emit_bf16:
            outs[1][...] = o.astype(jnp.bfloat16)

    out_specs = [pl.BlockSpec((BN, 128), lambda i: (i, 0))]
    out_shape = [jax.ShapeDtypeStruct((n, 128), jnp.float32)]
    if emit_bf16:
        out_specs.append(pl.BlockSpec((BN, 128), lambda i: (i, 0)))
        out_shape.append(jax.ShapeDtypeStruct((n, 128), jnp.bfloat16))

    return pl.pallas_call(
        body,
        grid=(grid,),
        in_specs=[
            pl.BlockSpec((c_in, BN, lanes), lambda i: (0, i, 0)),
            pl.BlockSpec((BN, d_in), lambda i: (i, 0)),
            pl.BlockSpec((2, BN, L), lambda i: (0, i, 0)),
            pl.BlockSpec((c_in * lanes, 128), lambda i: (0, 0)),
            pl.BlockSpec((d_in, 128), lambda i: (0, 0)),
            pl.BlockSpec((1, 128), lambda i: (0, 0)),
        ],
        out_specs=out_specs,
        out_shape=out_shape,
    )(agg, h, deg, wl, wr, bl)


def kernel(x, edge_index, Wl1, bl1, Wr1, Wl2, bl2, Wr2, Wl3, bl3, Wr3):
    n = x.shape[0]
    e = edge_index.shape[1]
    hid = Wl1.shape[1]

    unit = 2 * NTILES * SPB * IB
    e_pad = -(-e // unit) * unit
    eb = e_pad // IB
    pad = e_pad - e
    src_p = jnp.concatenate([edge_index[0], jnp.zeros((pad,), jnp.int32)])
    dst_p = jnp.concatenate([edge_index[1], jnp.full((pad,), n, jnp.int32)])
    srcb = src_p.reshape(eb, IB)
    dstb = dst_p.reshape(eb, IB)
    n_acc = n + L

    idx4, idx2 = _idx_prep(srcb, eb)
    zeros_hbm = jnp.zeros((ZB, L), jnp.float32)
    zeros_hbm_b = jnp.zeros((ZB, 2 * L), jnp.bfloat16)
    ones_hbm = jnp.ones((IB, L), jnp.float32)

    d_pad = 2 * L
    x_pad = jnp.pad(x, ((0, 0), (0, d_pad - x.shape[1])))
    wl1p = jnp.pad(Wl1, ((0, d_pad - Wl1.shape[0]), (0, 0)))
    wr1p = jnp.pad(Wr1, ((0, d_pad - Wr1.shape[0]), (0, 0)))

    sc_l1 = _sc_agg_build(n, n_acc, eb, 2, 1, True)
    sc_conv = _sc_agg_build(n, n_acc, eb, 4, 2, False,
                            lanes=2 * L, dtype=jnp.bfloat16)

    agg1f, degf = sc_l1(x_pad.reshape(n * 2, L), idx2, dstb, zeros_hbm, ones_hbm)
    agg1 = agg1f.reshape(2, n, L)
    deg = degf.reshape(2, n, L)
    h1, h1b = _mlp(agg1, x_pad, deg, wl1p, wr1p, bl1.reshape(1, hid), n, True)

    agg2 = sc_conv(h1b.reshape(n * 4, 2 * L), idx4, dstb, zeros_hbm_b, ones_hbm)[0]
    h2, h2b = _mlp(agg2.reshape(4, n, 2 * L), h1, deg, Wl2, Wr2,
                   bl2.reshape(1, hid), n, True)

    agg3 = sc_conv(h2b.reshape(n * 4, 2 * L), idx4, dstb, zeros_hbm_b, ones_hbm)[0]
    (h3,) = _mlp(agg3.reshape(4, n, 2 * L), h2, deg, Wl3, Wr3,
                 bl3.reshape(1, hid), n, False)
    return h3


# speedup vs baseline: 4.1569x; 1.0004x over previous
import functools

import jax
import jax.numpy as jnp
from jax import lax
from jax.experimental import pallas as pl
from jax.experimental.pallas import tpu as pltpu
from jax.experimental.pallas import tpu_sc as plsc

L = 16
IB = 128
SPB = 8
NTILES = 16
WB = 1000
ZB = 250


def _sc_agg_build(n, n_acc, eb, c_total, chunks_per_core, with_deg,
                  lanes=L, dtype=jnp.float32):
    blocks = eb // SPB
    per_tile = blocks // NTILES
    deg_per_tile = blocks // (2 * NTILES)
    rows_per_tile = n // NTILES
    n_sub = rows_per_tile // WB
    n_sub_z = rows_per_tile // ZB

    mesh = plsc.VectorSubcoreMesh(core_axis_name="c", subcore_axis_name="s")

    out_type = [jax.ShapeDtypeStruct((c_total, n, lanes), dtype)]
    if with_deg:
        out_type.append(jax.ShapeDtypeStruct((2, n, L), jnp.float32))

    scratch = [
        pltpu.VMEM_SHARED((n_acc, lanes), dtype),
        pltpu.VMEM((SPB, IB, lanes), dtype),
        pltpu.VMEM((SPB, IB, lanes), dtype),
        pltpu.VMEM((SPB, IB), jnp.int32),
        pltpu.VMEM((SPB, IB), jnp.int32),
        pltpu.VMEM((SPB, IB), jnp.int32),
        pltpu.VMEM((SPB, IB), jnp.int32),
        pltpu.VMEM((IB, L), jnp.float32),
        pltpu.VMEM((ZB, lanes), dtype),
        pltpu.SemaphoreType.DMA,
        pltpu.SemaphoreType.DMA,
    ]

    def body(hflat, gidx, dstb, zeros_hbm, ones_hbm, *refs):
        if with_deg:
            agg_out, deg_out = refs[0], refs[1]
            scr = refs[2:]
        else:
            agg_out = refs[0]
            deg_out = None
            scr = refs[1:]
        acc, r0v, r1v, g0v, g1v, d0v, d1v, ones_v, zeros_v, sem0, sem1 = scr
        bufs = ((r0v, g0v, d0v, sem0), (r1v, g1v, d1v, sem1))

        ci = lax.axis_index("c")
        sid = lax.axis_index("s")
        my_rows = sid * rows_per_tile

        pltpu.sync_copy(zeros_hbm, zeros_v)
        if with_deg:
            pltpu.sync_copy(ones_hbm, ones_v)

        def zero_acc():
            for k in range(n_sub_z):
                pltpu.sync_copy(zeros_v, acc.at[pl.ds(my_rows + k * ZB, ZB)])

        def writeout(out_ref, out_major):
            for k in range(n_sub):
                pltpu.sync_copy(acc.at[pl.ds(my_rows + k * WB, WB)],
                                out_ref.at[out_major, pl.ds(my_rows + k * WB, WB)])

        def fire(r0, buf, chunk):
            rows, gi, di, sem = buf
            pltpu.sync_copy(gidx.at[pl.ds(chunk * eb + r0, SPB)], gi)
            pltpu.sync_copy(dstb.at[pl.ds(r0, SPB)], di)
            for j in range(SPB):
                pltpu.async_copy(hflat.at[gi.at[j]], rows.at[j], sem)

        def drain_scatter(buf):
            rows, gi, di, sem = buf
            for j in range(SPB):
                pltpu.make_async_copy(hflat.at[gi.at[j]], rows.at[j], sem).wait()
            for j in range(SPB):
                pltpu.sync_copy(rows.at[j], acc.at[di.at[j]], add=True)

        for p in range(chunks_per_core):
            chunk = ci * chunks_per_core + p
            zero_acc()
            plsc.subcore_barrier()
            base = sid * per_tile * SPB

            fire(base, bufs[0], chunk)

            def pair_body(g, _, chunk=chunk, base=base):
                b0r = base + 2 * g * SPB
                fire(b0r + SPB, bufs[1], chunk)
                drain_scatter(bufs[0])

                @pl.when(2 * g + 2 < per_tile)
                def _():
                    fire(b0r + 2 * SPB, bufs[0], chunk)

                drain_scatter(bufs[1])
                return 0

            lax.fori_loop(0, per_tile // 2, pair_body, 0)
            plsc.subcore_barrier()
            writeout(agg_out, chunk)
            plsc.subcore_barrier()

        if with_deg:
            zero_acc()
            plsc.subcore_barrier()

            def deg_body(b, _):
                r0 = (ci * blocks // 2 + sid * deg_per_tile + b) * SPB
                pltpu.sync_copy(dstb.at[pl.ds(r0, SPB)], d0v)
                for j in range(SPB):
                    pltpu.sync_copy(ones_v, acc.at[d0v.at[j]], add=True)
                return 0

            lax.fori_loop(0, deg_per_tile, deg_body, 0)
            plsc.subcore_barrier()
            writeout(deg_out, ci)

    return pl.kernel(
        body, out_type=out_type, mesh=mesh, scratch_types=scratch,
        compiler_params=pltpu.CompilerParams(use_tc_tiling_on_sc=False))


def _idx_prep(srcb, eb):
    RB = 256
    grid = eb // RB

    def body(src_ref, i4_ref, i2_ref):
        s = src_ref[...]
        c4 = lax.broadcasted_iota(jnp.int32, (4, RB, IB), 0)
        i4_ref[...] = s[None] * 4 + c4
        c2 = lax.broadcasted_iota(jnp.int32, (2, RB, IB), 0)
        i2_ref[...] = s[None] * 2 + c2

    i4, i2 = pl.pallas_call(
        body,
        grid=(grid,),
        in_specs=[pl.BlockSpec((RB, IB), lambda i: (i, 0))],
        out_specs=[
            pl.BlockSpec((4, RB, IB), lambda i: (0, i, 0)),
            pl.BlockSpec((2, RB, IB), lambda i: (0, i, 0)),
        ],
        out_shape=[
            jax.ShapeDtypeStruct((4, eb, IB), jnp.int32),
            jax.ShapeDtypeStruct((2, eb, IB), jnp.int32),
        ],
    )(srcb)
    return i4.reshape(4 * eb, IB), i2.reshape(2 * eb, IB)


def _mlp(agg, h, deg, wl, wr, bl, n, emit_bf16):
    c_in, _, lanes = agg.shape
    d_in = h.shape[1]
    BN = 2000
    grid = n // BN

    def body(agg_ref, h_ref, deg_ref, wl_ref, wr_ref, bl_ref, *outs):
        deg_sum = deg_ref[0, :, 0:1] + deg_ref[1, :, 0:1]
        recip = 1.0 / jnp.maximum(deg_sum, 1.0)
        o = jnp.dot(h_ref[...], wr_ref[...], preferred_element_type=jnp.float32)
        for c in range(c_in):
            a = agg_ref[c].astype(jnp.float32) * recip
            o += jnp.dot(a, wl_ref[c * lanes:(c + 1) * lanes, :],
                         preferred_element_type=jnp.float32)
        o = jnp.maximum(o + bl_ref[...], 0.0)
        outs[0][...] = o
        if emit_bf16:
            outs[1][...] = o.astype(jnp.bfloat16)

    out_specs = [pl.BlockSpec((BN, 128), lambda i: (i, 0))]
    out_shape = [jax.ShapeDtypeStruct((n, 128), jnp.float32)]
    if emit_bf16:
        out_specs.append(pl.BlockSpec((BN, 128), lambda i: (i, 0)))
        out_shape.append(jax.ShapeDtypeStruct((n, 128), jnp.bfloat16))

    return pl.pallas_call(
        body,
        grid=(grid,),
        in_specs=[
            pl.BlockSpec((c_in, BN, lanes), lambda i: (0, i, 0)),
            pl.BlockSpec((BN, d_in), lambda i: (i, 0)),
            pl.BlockSpec((2, BN, L), lambda i: (0, i, 0)),
            pl.BlockSpec((c_in * lanes, 128), lambda i: (0, 0)),
            pl.BlockSpec((d_in, 128), lambda i: (0, 0)),
            pl.BlockSpec((1, 128), lambda i: (0, 0)),
        ],
        out_specs=out_specs,
        out_shape=out_shape,
    )(agg, h, deg, wl, wr, bl)


def kernel(x, edge_index, Wl1, bl1, Wr1, Wl2, bl2, Wr2, Wl3, bl3, Wr3):
    n = x.shape[0]
    e = edge_index.shape[1]
    hid = Wl1.shape[1]

    unit = 2 * NTILES * SPB * IB
    e_pad = -(-e // unit) * unit
    eb = e_pad // IB
    pad = e_pad - e
    src_p = jnp.concatenate([edge_index[0], jnp.zeros((pad,), jnp.int32)])
    dst_p = jnp.concatenate([edge_index[1], jnp.full((pad,), n, jnp.int32)])
    srcb = src_p.reshape(eb, IB)
    dstb = dst_p.reshape(eb, IB)
    n_acc = n + L

    idx4, idx2 = _idx_prep(srcb, eb)
    zeros_hbm = jnp.zeros((ZB, L), jnp.float32)
    zeros_hbm_b = jnp.zeros((ZB, 2 * L), jnp.bfloat16)
    ones_hbm = jnp.ones((IB, L), jnp.float32)

    d_pad = 2 * L
    x_pad = jnp.pad(x, ((0, 0), (0, d_pad - x.shape[1])))
    wl1p = jnp.pad(Wl1, ((0, d_pad - Wl1.shape[0]), (0, 0)))
    wr1p = jnp.pad(Wr1, ((0, d_pad - Wr1.shape[0]), (0, 0)))

    sc_l1 = _sc_agg_build(n, n_acc, eb, 2, 1, True)
    sc_conv = _sc_agg_build(n, n_acc, eb, 4, 2, False,
                            lanes=2 * L, dtype=jnp.bfloat16)

    agg1, deg = sc_l1(x_pad.reshape(n * 2, L), idx2, dstb, zeros_hbm, ones_hbm)
    h1, h1b = _mlp(agg1, x_pad, deg, wl1p, wr1p, bl1.reshape(1, hid), n, True)

    agg2 = sc_conv(h1b.reshape(n * 4, 2 * L), idx4, dstb, zeros_hbm_b, ones_hbm)[0]
    h2, h2b = _mlp(agg2, h1, deg, Wl2, Wr2, bl2.reshape(1, hid), n, True)

    agg3 = sc_conv(h2b.reshape(n * 4, 2 * L), idx4, dstb, zeros_hbm_b, ones_hbm)[0]
    (h3,) = _mlp(agg3, h2, deg, Wl3, Wr3, bl3.reshape(1, hid), n, False)
    return h3
